# baseline (speedup 1.0000x reference)
import jax
import jax.numpy as jnp
import numpy as np
from jax import lax
from jax.experimental import pallas as pl
from jax.experimental.pallas import tpu as pltpu
from jax.experimental.pallas import tpu_sc as plsc

NUM_BINS = 256
HIST = NUM_BINS * NUM_BINS
NT = 64
NROWS = 32768
NC = 2
NS = 16
LANES = 16
EPT = NROWS // NS
TPP = 8
NPASS = NT // (NC * TPP)
NFIRE = TPP * EPT // 128
SPMEM_WORDS = TPP * HIST
ZWORDS = 4096

CLIP = np.float32(1.0 - 1e-8)


def _bins(v):
    x = jnp.minimum(jnp.maximum(v, np.float32(0.0)), CLIP) * np.float32(256.0)
    k = x.astype(jnp.int32)
    b = jnp.where(k.astype(jnp.float32) == x, k - 1, k)
    return jnp.clip(b, 0, NUM_BINS - 1)


def _binify_body(yp_ref, yt_ref, out_ref):
    yp = yp_ref[...].reshape(EPT, NT)
    yt = yt_ref[...].reshape(EPT, NT)
    idx = (_bins(yp) * NUM_BINS + _bins(yt)
           + (lax.broadcasted_iota(jnp.int32, (EPT, NT), 1)
              & (TPP - 1)) * HIST)
    out_ref[...] = idx.T.reshape(NT, 1, EPT // 128, 128)


@jax.jit
def _tc_binify(yp3, yt3):
    return pl.pallas_call(
        _binify_body,
        grid=(NS,),
        in_specs=[pl.BlockSpec((1, EPT, NT), lambda s: (s, 0, 0))] * 2,
        out_specs=pl.BlockSpec((NT, 1, EPT // 128, 128),
                               lambda s: (0, s, 0, 0)),
        out_shape=jax.ShapeDtypeStruct((NT, NS, EPT // 128, 128), jnp.int32),
    )(yp3, yt3)


def _sc_body(idx_hbm, out_hbm, idxvm, ones_v, zeros_v, wbuf, hist_sh,
             isem, zsem, ssem, wsem):
    c = lax.axis_index("c")
    s = lax.axis_index("s")
    zstripe = TPP * HIST // NS

    for i in range(128 // LANES):
        ones_v[pl.ds(i * LANES, LANES)] = jnp.full((LANES,), 1.0, jnp.float32)

    def zfill(i, _):
        zeros_v[pl.ds(i * LANES, LANES)] = jnp.zeros((LANES,), jnp.float32)
        return 0

    lax.fori_loop(0, ZWORDS // LANES, zfill, 0)

    for p in range(NPASS):
        t0 = c * 32 + p * TPP
        if p:
            plsc.subcore_barrier()

        in_i = pltpu.async_copy(
            idx_hbm.at[pl.ds(t0, TPP), s, :, :], idxvm, isem)

        def zslot(i, _):
            pltpu.async_copy(
                zeros_v, hist_sh.at[pl.ds(s * zstripe + i * ZWORDS, ZWORDS)],
                zsem)
            return 0

        lax.fori_loop(0, zstripe // ZWORDS, zslot, 0)

        def zdrain(i, _):
            pltpu.make_async_copy(
                zeros_v, hist_sh.at[pl.ds(s * zstripe + i * ZWORDS, ZWORDS)],
                zsem).wait()
            return 0

        lax.fori_loop(0, zstripe // ZWORDS, zdrain, 0)
        plsc.subcore_barrier()

        in_i.wait()

        def fire(q, _):
            pltpu.async_copy(ones_v, hist_sh.at[idxvm.at[q >> 4, q & 15]],
                             ssem, add=True)
            return 0

        lax.fori_loop(0, NFIRE, fire, 0)

        def sdrain(q, _):
            pltpu.make_async_copy(ones_v,
                                  hist_sh.at[idxvm.at[q >> 4, q & 15]],
                                  ssem).wait()
            return 0

        lax.fori_loop(0, NFIRE, sdrain, 0)

        plsc.subcore_barrier()

        base = (s >> 1) * HIST + (s & 1) * (HIST // 2)

        def wrow(r, _):
            pltpu.async_copy(hist_sh.at[pl.ds(base + r * NUM_BINS, NUM_BINS)],
                             wbuf.at[r], wsem)
            return 0

        lax.fori_loop(0, NUM_BINS // 2, wrow, 0)

        def wdrain(r, _):
            pltpu.make_async_copy(
                hist_sh.at[pl.ds(base + r * NUM_BINS, NUM_BINS)],
                wbuf.at[r], wsem).wait()
            return 0

        lax.fori_loop(0, NUM_BINS // 2, wdrain, 0)
        pltpu.sync_copy(
            wbuf,
            out_hbm.at[t0 + (s >> 1),
                       pl.ds((s & 1) * (NUM_BINS // 2), NUM_BINS // 2), :])


@jax.jit
def _sc_hist(idx4):
    mesh = plsc.VectorSubcoreMesh(core_axis_name="c", subcore_axis_name="s",
                                  num_cores=NC, num_subcores=NS)
    return pl.kernel(
        _sc_body,
        out_type=jax.ShapeDtypeStruct((NT, NUM_BINS, NUM_BINS), jnp.float32),
        mesh=mesh,
        scratch_types=[
            pltpu.VMEM((TPP, EPT // 128, 128), jnp.int32),
            pltpu.VMEM((128,), jnp.float32),
            pltpu.VMEM((ZWORDS,), jnp.float32),
            pltpu.VMEM((NUM_BINS // 2, NUM_BINS), jnp.float32),
            pltpu.VMEM_SHARED((SPMEM_WORDS,), jnp.float32),
            pltpu.SemaphoreType.DMA,
            pltpu.SemaphoreType.DMA,
            pltpu.SemaphoreType.DMA,
            pltpu.SemaphoreType.DMA,
        ],
    )(idx4)


TRACKS_PER_STEP = 8


def _reduce_body(j_ref, pred_ref, targ_ref):
    j = j_ref[...]
    pred_ref[...] = jnp.sum(j, axis=2)
    targ_ref[...] = jnp.sum(j, axis=1)


@jax.jit
def _tc_reduce(joint3):
    return pl.pallas_call(
        _reduce_body,
        grid=(NT // TRACKS_PER_STEP,),
        in_specs=[pl.BlockSpec((TRACKS_PER_STEP, NUM_BINS, NUM_BINS),
                               lambda t: (t, 0, 0))],
        out_specs=[pl.BlockSpec((TRACKS_PER_STEP, NUM_BINS), lambda t: (t, 0)),
                   pl.BlockSpec((TRACKS_PER_STEP, NUM_BINS), lambda t: (t, 0))],
        out_shape=[jax.ShapeDtypeStruct((NT, NUM_BINS), jnp.float32)] * 2,
    )(joint3)


def kernel(y_pred, y_true):
    yp = y_pred.astype(jnp.float32)
    yt = y_true.astype(jnp.float32)
    idx4 = _tc_binify(yp, yt)
    joint3 = _sc_hist(idx4)
    pred_hist, target_hist = _tc_reduce(joint3)
    return pred_hist, target_hist, joint3


# speedup vs baseline: 1.0226x; 1.0226x over previous
import jax
import jax.numpy as jnp
import numpy as np
from jax import lax
from jax.experimental import pallas as pl
from jax.experimental.pallas import tpu as pltpu
from jax.experimental.pallas import tpu_sc as plsc

NUM_BINS = 256
HIST = NUM_BINS * NUM_BINS
NT = 64
NROWS = 32768
NC = 2
NS = 16
LANES = 16
EPT = NROWS // NS
TPP = 8
NPASS = NT // (NC * TPP)
NFIRE = TPP * EPT // 128
SPMEM_WORDS = TPP * HIST
ZWORDS = 4096

CLIP = np.float32(1.0 - 1e-8)


def _bins(v):
    x = jnp.minimum(jnp.maximum(v, np.float32(0.0)), CLIP) * np.float32(256.0)
    k = x.astype(jnp.int32)
    b = jnp.where(k.astype(jnp.float32) == x, k - 1, k)
    return jnp.clip(b, 0, NUM_BINS - 1)


def _binify_body(yp_ref, yt_ref, out_ref):
    idx = (_bins(yp_ref[...]) * NUM_BINS + _bins(yt_ref[...])
           + (lax.broadcasted_iota(jnp.int32, (EPT, NT), 1)
              & (TPP - 1)) * HIST)
    out_ref[...] = idx.T.reshape(NT, 1, EPT // 128, 128)


@jax.jit
def _tc_binify(yp, yt):
    return pl.pallas_call(
        _binify_body,
        grid=(NS,),
        in_specs=[pl.BlockSpec((EPT, NT), lambda s: (s, 0))] * 2,
        out_specs=pl.BlockSpec((NT, 1, EPT // 128, 128),
                               lambda s: (0, s, 0, 0)),
        out_shape=jax.ShapeDtypeStruct((NT, NS, EPT // 128, 128), jnp.int32),
    )(yp, yt)


def _sc_body(idx_hbm, out_hbm, idxvm, ones_v, zeros_v, wbuf, hist_sh,
             isem, zsem, ssem, wsem):
    c = lax.axis_index("c")
    s = lax.axis_index("s")
    zstripe = TPP * HIST // NS

    def ofill(i, _):
        ones_v[pl.ds(i * LANES, LANES)] = jnp.full((LANES,), 1.0, jnp.float32)
        return 0

    lax.fori_loop(0, 128 // LANES, ofill, 0)

    def zfill(i, _):
        zeros_v[pl.ds(i * LANES, LANES)] = jnp.zeros((LANES,), jnp.float32)
        return 0

    lax.fori_loop(0, ZWORDS // LANES, zfill, 0)

    for p in range(NPASS):
        t0 = c * 32 + p * TPP
        if p:
            plsc.subcore_barrier()

        in_i = pltpu.async_copy(
            idx_hbm.at[pl.ds(t0, TPP), s, :, :], idxvm, isem)

        def zslot(i, _):
            pltpu.async_copy(
                zeros_v, hist_sh.at[pl.ds(s * zstripe + i * ZWORDS, ZWORDS)],
                zsem)
            return 0

        lax.fori_loop(0, zstripe // ZWORDS, zslot, 0)

        def zdrain(i, _):
            pltpu.make_async_copy(
                zeros_v, hist_sh.at[pl.ds(s * zstripe + i * ZWORDS, ZWORDS)],
                zsem).wait()
            return 0

        lax.fori_loop(0, zstripe // ZWORDS, zdrain, 0)
        plsc.subcore_barrier()

        in_i.wait()

        def fire(q, _):
            pltpu.async_copy(ones_v, hist_sh.at[idxvm.at[q >> 4, q & 15]],
                             ssem, add=True)
            return 0

        lax.fori_loop(0, NFIRE, fire, 0)

        def sdrain(q, _):
            pltpu.make_async_copy(ones_v,
                                  hist_sh.at[idxvm.at[q >> 4, q & 15]],
                                  ssem).wait()
            return 0

        lax.fori_loop(0, NFIRE, sdrain, 0)

        plsc.subcore_barrier()

        base = (s >> 1) * HIST + (s & 1) * (HIST // 2)

        def wrow(r, _):
            pltpu.async_copy(hist_sh.at[pl.ds(base + r * NUM_BINS, NUM_BINS)],
                             wbuf.at[r], wsem)
            return 0

        lax.fori_loop(0, NUM_BINS // 2, wrow, 0)

        def wdrain(r, _):
            pltpu.make_async_copy(
                hist_sh.at[pl.ds(base + r * NUM_BINS, NUM_BINS)],
                wbuf.at[r], wsem).wait()
            return 0

        lax.fori_loop(0, NUM_BINS // 2, wdrain, 0)
        pltpu.sync_copy(
            wbuf,
            out_hbm.at[t0 + (s >> 1),
                       pl.ds((s & 1) * (NUM_BINS // 2), NUM_BINS // 2), :])


@jax.jit
def _sc_hist(idx4):
    mesh = plsc.VectorSubcoreMesh(core_axis_name="c", subcore_axis_name="s",
                                  num_cores=NC, num_subcores=NS)
    return pl.kernel(
        _sc_body,
        out_type=jax.ShapeDtypeStruct((NT, NUM_BINS, NUM_BINS), jnp.float32),
        mesh=mesh,
        scratch_types=[
            pltpu.VMEM((TPP, EPT // 128, 128), jnp.int32),
            pltpu.VMEM((128,), jnp.float32),
            pltpu.VMEM((ZWORDS,), jnp.float32),
            pltpu.VMEM((NUM_BINS // 2, NUM_BINS), jnp.float32),
            pltpu.VMEM_SHARED((SPMEM_WORDS,), jnp.float32),
            pltpu.SemaphoreType.DMA,
            pltpu.SemaphoreType.DMA,
            pltpu.SemaphoreType.DMA,
            pltpu.SemaphoreType.DMA,
        ],
    )(idx4)


TRACKS_PER_STEP = 8


def _reduce_body(j_ref, pred_ref, targ_ref):
    j = j_ref[...]
    pred_ref[...] = jnp.sum(j, axis=2)
    targ_ref[...] = jnp.sum(j, axis=1)


@jax.jit
def _tc_reduce(joint3):
    return pl.pallas_call(
        _reduce_body,
        grid=(NT // TRACKS_PER_STEP,),
        in_specs=[pl.BlockSpec((TRACKS_PER_STEP, NUM_BINS, NUM_BINS),
                               lambda t: (t, 0, 0))],
        out_specs=[pl.BlockSpec((TRACKS_PER_STEP, NUM_BINS), lambda t: (t, 0)),
                   pl.BlockSpec((TRACKS_PER_STEP, NUM_BINS), lambda t: (t, 0))],
        out_shape=[jax.ShapeDtypeStruct((NT, NUM_BINS), jnp.float32)] * 2,
    )(joint3)


def kernel(y_pred, y_true):
    yp = y_pred.astype(jnp.float32).reshape(-1, NT)
    yt = y_true.astype(jnp.float32).reshape(-1, NT)
    idx4 = _tc_binify(yp, yt)
    joint3 = _sc_hist(idx4)
    pred_hist, target_hist = _tc_reduce(joint3)
    return pred_hist, target_hist, joint3


# speedup vs baseline: 1.3871x; 1.3565x over previous
import jax
import jax.numpy as jnp
import numpy as np
from jax import lax
from jax.experimental import pallas as pl
from jax.experimental.pallas import tpu as pltpu
from jax.experimental.pallas import tpu_sc as plsc

NUM_BINS = 256
HIST = NUM_BINS * NUM_BINS
NT = 64
NROWS = 32768
NC = 2
NS = 16
LANES = 16
EPT = NROWS // NS
TPP = 8
NPASS = NT // (NC * TPP)
NFIRE = TPP * EPT // 128
SPMEM_WORDS = TPP * HIST
ZWORDS = 4096

CLIP = np.float32(1.0 - 1e-8)


def _bins(v):
    x = jnp.minimum(jnp.maximum(v, np.float32(0.0)), CLIP) * np.float32(256.0)
    k = x.astype(jnp.int32)
    b = jnp.where(k.astype(jnp.float32) == x, k - 1, k)
    return jnp.clip(b, 0, NUM_BINS - 1)


def _binify_body(yp_ref, yt_ref, out_ref):
    yp = yp_ref[...].reshape(NT, EPT)
    yt = yt_ref[...].reshape(NT, EPT)
    idx = (_bins(yp) * NUM_BINS + _bins(yt)
           + (lax.broadcasted_iota(jnp.int32, (NT, EPT), 0)
              & (TPP - 1)) * HIST)
    out_ref[...] = idx.reshape(NT, 1, EPT // 128, 128)


@jax.jit
def _tc_binify(ypt, ytt):
    return pl.pallas_call(
        _binify_body,
        grid=(NS,),
        in_specs=[pl.BlockSpec((1, NT, EPT), lambda s: (s, 0, 0))] * 2,
        out_specs=pl.BlockSpec((NT, 1, EPT // 128, 128),
                               lambda s: (0, s, 0, 0)),
        out_shape=jax.ShapeDtypeStruct((NT, NS, EPT // 128, 128), jnp.int32),
    )(ypt, ytt)


def _sc_body(idx_hbm, out_hbm, idxvm, ones_v, zeros_v, wbuf, hist_sh,
             isem, zsem, ssem, wsem):
    c = lax.axis_index("c")
    s = lax.axis_index("s")
    zstripe = TPP * HIST // NS

    def ofill(i, _):
        ones_v[pl.ds(i * LANES, LANES)] = jnp.full((LANES,), 1.0, jnp.float32)
        return 0

    lax.fori_loop(0, 128 // LANES, ofill, 0)

    def zfill(i, _):
        zeros_v[pl.ds(i * LANES, LANES)] = jnp.zeros((LANES,), jnp.float32)
        return 0

    lax.fori_loop(0, ZWORDS // LANES, zfill, 0)

    for p in range(NPASS):
        t0 = c * 32 + p * TPP
        if p:
            plsc.subcore_barrier()

        in_i = pltpu.async_copy(
            idx_hbm.at[pl.ds(t0, TPP), s, :, :], idxvm, isem)

        def zslot(i, _):
            pltpu.async_copy(
                zeros_v, hist_sh.at[pl.ds(s * zstripe + i * ZWORDS, ZWORDS)],
                zsem)
            return 0

        lax.fori_loop(0, zstripe // ZWORDS, zslot, 0)

        def zdrain(i, _):
            pltpu.make_async_copy(
                zeros_v, hist_sh.at[pl.ds(s * zstripe + i * ZWORDS, ZWORDS)],
                zsem).wait()
            return 0

        lax.fori_loop(0, zstripe // ZWORDS, zdrain, 0)
        plsc.subcore_barrier()

        in_i.wait()

        def fire(q, _):
            pltpu.async_copy(ones_v, hist_sh.at[idxvm.at[q >> 4, q & 15]],
                             ssem, add=True)
            return 0

        lax.fori_loop(0, NFIRE, fire, 0)

        def sdrain(q, _):
            pltpu.make_async_copy(ones_v,
                                  hist_sh.at[idxvm.at[q >> 4, q & 15]],
                                  ssem).wait()
            return 0

        lax.fori_loop(0, NFIRE, sdrain, 0)

        plsc.subcore_barrier()

        base = (s >> 1) * HIST + (s & 1) * (HIST // 2)

        def wrow(r, _):
            pltpu.async_copy(hist_sh.at[pl.ds(base + r * NUM_BINS, NUM_BINS)],
                             wbuf.at[r], wsem)
            return 0

        lax.fori_loop(0, NUM_BINS // 2, wrow, 0)

        def wdrain(r, _):
            pltpu.make_async_copy(
                hist_sh.at[pl.ds(base + r * NUM_BINS, NUM_BINS)],
                wbuf.at[r], wsem).wait()
            return 0

        lax.fori_loop(0, NUM_BINS // 2, wdrain, 0)
        pltpu.sync_copy(
            wbuf,
            out_hbm.at[t0 + (s >> 1),
                       pl.ds((s & 1) * (NUM_BINS // 2), NUM_BINS // 2), :])


@jax.jit
def _sc_hist(idx4):
    mesh = plsc.VectorSubcoreMesh(core_axis_name="c", subcore_axis_name="s",
                                  num_cores=NC, num_subcores=NS)
    return pl.kernel(
        _sc_body,
        out_type=jax.ShapeDtypeStruct((NT, NUM_BINS, NUM_BINS), jnp.float32),
        mesh=mesh,
        scratch_types=[
            pltpu.VMEM((TPP, EPT // 128, 128), jnp.int32),
            pltpu.VMEM((128,), jnp.float32),
            pltpu.VMEM((ZWORDS,), jnp.float32),
            pltpu.VMEM((NUM_BINS // 2, NUM_BINS), jnp.float32),
            pltpu.VMEM_SHARED((SPMEM_WORDS,), jnp.float32),
            pltpu.SemaphoreType.DMA,
            pltpu.SemaphoreType.DMA,
            pltpu.SemaphoreType.DMA,
            pltpu.SemaphoreType.DMA,
        ],
    )(idx4)


TRACKS_PER_STEP = 8


def _reduce_body(j_ref, pred_ref, targ_ref):
    j = j_ref[...]
    pred_ref[...] = jnp.sum(j, axis=2)
    targ_ref[...] = jnp.sum(j, axis=1)


@jax.jit
def _tc_reduce(joint3):
    return pl.pallas_call(
        _reduce_body,
        grid=(NT // TRACKS_PER_STEP,),
        in_specs=[pl.BlockSpec((TRACKS_PER_STEP, NUM_BINS, NUM_BINS),
                               lambda t: (t, 0, 0))],
        out_specs=[pl.BlockSpec((TRACKS_PER_STEP, NUM_BINS), lambda t: (t, 0)),
                   pl.BlockSpec((TRACKS_PER_STEP, NUM_BINS), lambda t: (t, 0))],
        out_shape=[jax.ShapeDtypeStruct((NT, NUM_BINS), jnp.float32)] * 2,
    )(joint3)


def kernel(y_pred, y_true):
    yp = jnp.transpose(y_pred.astype(jnp.float32), (0, 2, 1))
    yt = jnp.transpose(y_true.astype(jnp.float32), (0, 2, 1))
    idx4 = _tc_binify(yp, yt)
    joint3 = _sc_hist(idx4)
    pred_hist, target_hist = _tc_reduce(joint3)
    return pred_hist, target_hist, joint3
